# baseline (speedup 1.0000x reference)
import functools

import jax
import jax.numpy as jnp
from jax import lax
from jax.experimental import pallas as pl
from jax.experimental.pallas import tpu as pltpu
from jax.experimental.pallas import tpu_sc as plsc

NC = 2
NS = 16
NW = NC * NS

CH = 128

_MESH = plsc.VectorSubcoreMesh(core_axis_name="c", subcore_axis_name="s")


def _zero_init(zf_h, rows, acc_sh, row0, npiece):
    pltpu.sync_copy(zf_h, rows)

    @pl.loop(0, npiece)
    def _(j):
        pltpu.sync_copy(rows, acc_sh.at[pl.ds(row0 + j * CH, CH)])


def _copy_out(acc_sh, rows, out_h, c, row0, npiece):
    @pl.loop(0, npiece)
    def _(j):
        r = row0 + j * CH
        pltpu.sync_copy(acc_sh.at[pl.ds(r, CH)], rows)
        pltpu.sync_copy(rows, out_h.at[c, pl.ds(r, CH)])


def _make_sc_aggregate(n_pad, d, nch):
    rpt = n_pad // NS
    npiece = rpt // CH

    def body(table_h, eidx_h, zf_h, agg_h, acc_sh,
             eidx0, eidx1, rows0, rows1, sg0, sg1, ss0, ss1):
        c = lax.axis_index("c")
        s = lax.axis_index("s")
        w = c * NS + s
        row0 = s * rpt
        base = w * nch

        _zero_init(zf_h, rows0, acc_sh, row0, npiece)
        plsc.subcore_barrier()

        @pl.loop(0, nch // 2)
        def _(kk):
            k0 = base + kk * 2

            @pl.when(kk > 0)
            def _():
                pltpu.make_async_copy(rows0, acc_sh.at[eidx0.at[1]], ss0).wait()

            pltpu.sync_copy(eidx_h.at[k0], eidx0)
            g0 = pltpu.async_copy(table_h.at[eidx0.at[0]], rows0, sg0)

            @pl.when(kk > 0)
            def _():
                pltpu.make_async_copy(rows1, acc_sh.at[eidx1.at[1]], ss1).wait()

            g0.wait()
            pltpu.async_copy(rows0, acc_sh.at[eidx0.at[1]], ss0, add=True)

            pltpu.sync_copy(eidx_h.at[k0 + 1], eidx1)
            g1 = pltpu.async_copy(table_h.at[eidx1.at[0]], rows1, sg1)
            g1.wait()
            pltpu.async_copy(rows1, acc_sh.at[eidx1.at[1]], ss1, add=True)

        pltpu.make_async_copy(rows0, acc_sh.at[eidx0.at[1]], ss0).wait()
        pltpu.make_async_copy(rows1, acc_sh.at[eidx1.at[1]], ss1).wait()
        plsc.subcore_barrier()

        _copy_out(acc_sh, rows0, agg_h, c, row0, npiece)

    return pl.kernel(
        body,
        out_type=jax.ShapeDtypeStruct((NC, n_pad, d), jnp.float32),
        mesh=_MESH,
        scratch_types=[
            pltpu.VMEM_SHARED((n_pad, d), jnp.float32),
            pltpu.VMEM((2, CH), jnp.int32),
            pltpu.VMEM((2, CH), jnp.int32),
            pltpu.VMEM((CH, d), jnp.float32),
            pltpu.VMEM((CH, d), jnp.float32),
            pltpu.SemaphoreType.DMA,
            pltpu.SemaphoreType.DMA,
            pltpu.SemaphoreType.DMA,
            pltpu.SemaphoreType.DMA,
        ],
    )


def _make_sc_count(n_pad, d, nch):
    rpt = n_pad // NS
    npiece = rpt // CH

    def body(eidx_h, zf_h, on_h, cnt_h, cnt_sh,
             eidx0, eidx1, rows, ones_v, ss0, ss1):
        c = lax.axis_index("c")
        s = lax.axis_index("s")
        w = c * NS + s
        row0 = s * rpt
        base = w * nch

        _zero_init(zf_h, rows, cnt_sh, row0, npiece)
        pltpu.sync_copy(on_h, ones_v)
        plsc.subcore_barrier()

        @pl.loop(0, nch // 2)
        def _(kk):
            k0 = base + kk * 2

            @pl.when(kk > 0)
            def _():
                pltpu.make_async_copy(ones_v, cnt_sh.at[eidx0.at[1]], ss0).wait()

            pltpu.sync_copy(eidx_h.at[k0], eidx0)
            pltpu.async_copy(ones_v, cnt_sh.at[eidx0.at[1]], ss0, add=True)

            @pl.when(kk > 0)
            def _():
                pltpu.make_async_copy(ones_v, cnt_sh.at[eidx1.at[1]], ss1).wait()

            pltpu.sync_copy(eidx_h.at[k0 + 1], eidx1)
            pltpu.async_copy(ones_v, cnt_sh.at[eidx1.at[1]], ss1, add=True)

        pltpu.make_async_copy(ones_v, cnt_sh.at[eidx0.at[1]], ss0).wait()
        pltpu.make_async_copy(ones_v, cnt_sh.at[eidx1.at[1]], ss1).wait()
        plsc.subcore_barrier()

        _copy_out(cnt_sh, rows, cnt_h, c, row0, npiece)

    return pl.kernel(
        body,
        out_type=jax.ShapeDtypeStruct((NC, n_pad, d), jnp.float32),
        mesh=_MESH,
        scratch_types=[
            pltpu.VMEM_SHARED((n_pad, d), jnp.float32),
            pltpu.VMEM((2, CH), jnp.int32),
            pltpu.VMEM((2, CH), jnp.int32),
            pltpu.VMEM((CH, d), jnp.float32),
            pltpu.VMEM((CH, d), jnp.float32),
            pltpu.SemaphoreType.DMA,
            pltpu.SemaphoreType.DMA,
        ],
    )


def _combine_body(agg_ref, cnt_ref, h_ref, wl_ref, wr_ref, b_ref, out_ref, *,
                  relu):
    agg = agg_ref[0] + agg_ref[1]
    cnt = cnt_ref[0, :, 0:1] + cnt_ref[1, :, 0:1]
    mean = agg / jnp.maximum(cnt, 1.0)
    acc = jnp.dot(mean, wl_ref[...], preferred_element_type=jnp.float32,
                  precision=lax.Precision.HIGHEST)
    acc += jnp.dot(h_ref[...], wr_ref[...], preferred_element_type=jnp.float32,
                   precision=lax.Precision.HIGHEST)
    acc += b_ref[...]
    out_ref[...] = jnp.maximum(acc, 0.0) if relu else acc


def _combine(agg, cnt, h, wl, wr, b, relu, block):
    n, d = h.shape
    grid = (n // block,)
    return pl.pallas_call(
        functools.partial(_combine_body, relu=relu),
        grid=grid,
        in_specs=[
            pl.BlockSpec((NC, block, d), lambda i: (0, i, 0)),
            pl.BlockSpec((NC, block, d), lambda i: (0, i, 0)),
            pl.BlockSpec((block, d), lambda i: (i, 0)),
            pl.BlockSpec((d, d), lambda i: (0, 0)),
            pl.BlockSpec((d, d), lambda i: (0, 0)),
            pl.BlockSpec((1, d), lambda i: (0, 0)),
        ],
        out_specs=pl.BlockSpec((block, d), lambda i: (i, 0)),
        out_shape=jax.ShapeDtypeStruct((n, d), jnp.float32),
    )(agg, cnt, h, wl, wr, b.reshape(1, d))


@jax.jit
def kernel(x, edge_index, Wl0, Wr0, b0, Wl1, Wr1, b1):
    n, d = x.shape
    e = edge_index.shape[1]
    blk = NS * CH
    n_pad = ((n + blk - 1) // blk) * blk

    per_w = 2 * CH
    e_pad = ((e + NW * per_w - 1) // (NW * per_w)) * (NW * per_w)
    nch = e_pad // (NW * CH)
    pad = e_pad - e
    src = jnp.concatenate([edge_index[0], jnp.zeros((pad,), jnp.int32)])
    dst = jnp.concatenate(
        [edge_index[1], jnp.full((pad,), n_pad - 1, jnp.int32)])
    eidx = jnp.stack(
        [src.reshape(NW * nch, CH), dst.reshape(NW * nch, CH)], axis=1)

    zeros_feat = jnp.zeros((CH, d), jnp.float32)
    ones_rows = jnp.ones((CH, d), jnp.float32)

    cnt = _make_sc_count(n_pad, d, nch)(eidx, zeros_feat, ones_rows)
    agg_x = _make_sc_aggregate(n_pad, d, nch)(x, eidx, zeros_feat)
    h1 = _combine(agg_x, cnt, x, Wl0, Wr0, b0, relu=True, block=400)
    agg_h = _make_sc_aggregate(n_pad, d, nch)(h1, eidx, zeros_feat)
    out = _combine(agg_h, cnt, h1, Wl1, Wr1, b1, relu=False, block=400)
    return out


# speedup vs baseline: 2.3356x; 2.3356x over previous
import functools

import jax
import jax.numpy as jnp
from jax import lax
from jax.experimental import pallas as pl
from jax.experimental.pallas import tpu as pltpu
from jax.experimental.pallas import tpu_sc as plsc

NC = 2
NS = 16
NW = NC * NS

CH = 128

_MESH = plsc.VectorSubcoreMesh(core_axis_name="c", subcore_axis_name="s")


def _zero_init(zf_h, rows, acc_sh, row0, npiece):
    pltpu.sync_copy(zf_h, rows)

    @pl.loop(0, npiece)
    def _(j):
        pltpu.sync_copy(rows, acc_sh.at[pl.ds(row0 + j * CH, CH)])


def _copy_out(acc_sh, rows, out_h, c, row0, npiece):
    @pl.loop(0, npiece)
    def _(j):
        r = row0 + j * CH
        pltpu.sync_copy(acc_sh.at[pl.ds(r, CH)], rows)
        pltpu.sync_copy(rows, out_h.at[c, pl.ds(r, CH)])


def _make_sc_aggregate(n_pad, d, nch):
    rpt = n_pad // NS
    npiece = rpt // CH

    def body(table_h, eidx_h, zf_h, agg_h, acc_sh,
             eidx0, eidx1, rows0, rows1, sg0, sg1, ss0, ss1):
        c = lax.axis_index("c")
        s = lax.axis_index("s")
        w = c * NS + s
        row0 = s * rpt
        base = w * nch

        _zero_init(zf_h, rows0, acc_sh, row0, npiece)
        plsc.subcore_barrier()

        @pl.loop(0, nch // 2)
        def _(kk):
            k0 = base + kk * 2

            @pl.when(kk > 0)
            def _():
                pltpu.make_async_copy(rows0, acc_sh.at[eidx0.at[1]], ss0).wait()

            pltpu.sync_copy(eidx_h.at[k0], eidx0)
            g0 = pltpu.async_copy(table_h.at[eidx0.at[0]], rows0, sg0)

            @pl.when(kk > 0)
            def _():
                pltpu.make_async_copy(rows1, acc_sh.at[eidx1.at[1]], ss1).wait()

            g0.wait()
            pltpu.async_copy(rows0, acc_sh.at[eidx0.at[1]], ss0, add=True)

            pltpu.sync_copy(eidx_h.at[k0 + 1], eidx1)
            g1 = pltpu.async_copy(table_h.at[eidx1.at[0]], rows1, sg1)
            g1.wait()
            pltpu.async_copy(rows1, acc_sh.at[eidx1.at[1]], ss1, add=True)

        pltpu.make_async_copy(rows0, acc_sh.at[eidx0.at[1]], ss0).wait()
        pltpu.make_async_copy(rows1, acc_sh.at[eidx1.at[1]], ss1).wait()
        plsc.subcore_barrier()

        _copy_out(acc_sh, rows0, agg_h, c, row0, npiece)

    return pl.kernel(
        body,
        out_type=jax.ShapeDtypeStruct((NC, n_pad, d), jnp.float32),
        mesh=_MESH,
        scratch_types=[
            pltpu.VMEM_SHARED((n_pad, d), jnp.float32),
            pltpu.VMEM((2, CH), jnp.int32),
            pltpu.VMEM((2, CH), jnp.int32),
            pltpu.VMEM((CH, d), jnp.float32),
            pltpu.VMEM((CH, d), jnp.float32),
            pltpu.SemaphoreType.DMA,
            pltpu.SemaphoreType.DMA,
            pltpu.SemaphoreType.DMA,
            pltpu.SemaphoreType.DMA,
        ],
    )


def _make_sc_count(n_pad, d, nch):
    rpt = n_pad // NS
    npiece = rpt // CH

    def body(eidx_h, zf_h, on_h, cnt_h, cnt_sh,
             eidx0, eidx1, rows, ones_v, ss0, ss1):
        c = lax.axis_index("c")
        s = lax.axis_index("s")
        w = c * NS + s
        row0 = s * rpt
        base = w * nch

        _zero_init(zf_h, rows, cnt_sh, row0, npiece)
        pltpu.sync_copy(on_h, ones_v)
        plsc.subcore_barrier()

        @pl.loop(0, nch // 2)
        def _(kk):
            k0 = base + kk * 2

            @pl.when(kk > 0)
            def _():
                pltpu.make_async_copy(ones_v, cnt_sh.at[eidx0.at[1]], ss0).wait()

            pltpu.sync_copy(eidx_h.at[k0], eidx0)
            pltpu.async_copy(ones_v, cnt_sh.at[eidx0.at[1]], ss0, add=True)

            @pl.when(kk > 0)
            def _():
                pltpu.make_async_copy(ones_v, cnt_sh.at[eidx1.at[1]], ss1).wait()

            pltpu.sync_copy(eidx_h.at[k0 + 1], eidx1)
            pltpu.async_copy(ones_v, cnt_sh.at[eidx1.at[1]], ss1, add=True)

        pltpu.make_async_copy(ones_v, cnt_sh.at[eidx0.at[1]], ss0).wait()
        pltpu.make_async_copy(ones_v, cnt_sh.at[eidx1.at[1]], ss1).wait()
        plsc.subcore_barrier()

        _copy_out(cnt_sh, rows, cnt_h, c, row0, npiece)

    return pl.kernel(
        body,
        out_type=jax.ShapeDtypeStruct((NC, n_pad, d), jnp.float32),
        mesh=_MESH,
        scratch_types=[
            pltpu.VMEM_SHARED((n_pad, d), jnp.float32),
            pltpu.VMEM((2, CH), jnp.int32),
            pltpu.VMEM((2, CH), jnp.int32),
            pltpu.VMEM((CH, d), jnp.float32),
            pltpu.VMEM((CH, d), jnp.float32),
            pltpu.SemaphoreType.DMA,
            pltpu.SemaphoreType.DMA,
        ],
    )


def _combine_body(agg_ref, cnt_ref, h_ref, wl_ref, wr_ref, b_ref, out_ref, *,
                  relu):
    agg = agg_ref[0] + agg_ref[1]
    cnt = cnt_ref[0, :, 0:1] + cnt_ref[1, :, 0:1]
    mean = agg / jnp.maximum(cnt, 1.0)
    acc = jnp.dot(mean, wl_ref[...], preferred_element_type=jnp.float32,
                  precision=lax.Precision.HIGHEST)
    acc += jnp.dot(h_ref[...], wr_ref[...], preferred_element_type=jnp.float32,
                   precision=lax.Precision.HIGHEST)
    acc += b_ref[...]
    out_ref[...] = jnp.maximum(acc, 0.0) if relu else acc


def _combine(agg, cnt, h, wl, wr, b, relu, block):
    n, d = h.shape
    grid = (n // block,)
    return pl.pallas_call(
        functools.partial(_combine_body, relu=relu),
        grid=grid,
        in_specs=[
            pl.BlockSpec((NC, block, d), lambda i: (0, i, 0)),
            pl.BlockSpec((NC, block, d), lambda i: (0, i, 0)),
            pl.BlockSpec((block, d), lambda i: (i, 0)),
            pl.BlockSpec((d, d), lambda i: (0, 0)),
            pl.BlockSpec((d, d), lambda i: (0, 0)),
            pl.BlockSpec((1, d), lambda i: (0, 0)),
        ],
        out_specs=pl.BlockSpec((block, d), lambda i: (i, 0)),
        out_shape=jax.ShapeDtypeStruct((n, d), jnp.float32),
    )(agg, cnt, h, wl, wr, b.reshape(1, d))


@jax.jit
def kernel(x, edge_index, Wl0, Wr0, b0, Wl1, Wr1, b1):
    n, d = x.shape
    e = edge_index.shape[1]
    blk = NS * CH
    n_pad = ((n + blk - 1) // blk) * blk

    per_w = 2 * CH
    e_pad = ((e + NW * per_w - 1) // (NW * per_w)) * (NW * per_w)
    nch = e_pad // (NW * CH)
    pad = e_pad - e
    pad_ar = lax.iota(jnp.int32, pad)
    src = jnp.concatenate([edge_index[0], pad_ar % n])
    dst = jnp.concatenate([edge_index[1], n + pad_ar % (n_pad - n)])
    eidx = jnp.stack(
        [src.reshape(NW * nch, CH), dst.reshape(NW * nch, CH)], axis=1)

    zeros_feat = jnp.zeros((CH, d), jnp.float32)
    ones_rows = jnp.ones((CH, d), jnp.float32)

    cnt = _make_sc_count(n_pad, d, nch)(eidx, zeros_feat, ones_rows)
    agg_x = _make_sc_aggregate(n_pad, d, nch)(x, eidx, zeros_feat)
    h1 = _combine(agg_x, cnt, x, Wl0, Wr0, b0, relu=True, block=400)
    agg_h = _make_sc_aggregate(n_pad, d, nch)(h1, eidx, zeros_feat)
    out = _combine(agg_h, cnt, h1, Wl1, Wr1, b1, relu=False, block=400)
    return out


# speedup vs baseline: 2.5167x; 1.0775x over previous
import functools

import jax
import jax.numpy as jnp
from jax import lax
from jax.experimental import pallas as pl
from jax.experimental.pallas import tpu as pltpu
from jax.experimental.pallas import tpu_sc as plsc

NC = 2
NS = 16
NW = NC * NS

CH = 128
BLK = 10

_MESH = plsc.VectorSubcoreMesh(core_axis_name="c", subcore_axis_name="s")


def _zero_init(zf_h, rows, acc_sh, row0, npiece):
    pltpu.sync_copy(zf_h, rows)

    @pl.loop(0, npiece)
    def _(j):
        pltpu.sync_copy(rows, acc_sh.at[pl.ds(row0 + j * CH, CH)])


def _copy_out(acc_sh, rows, out_h, c, row0, npiece):
    @pl.loop(0, npiece)
    def _(j):
        r = row0 + j * CH
        pltpu.sync_copy(acc_sh.at[pl.ds(r, CH)], rows)
        pltpu.sync_copy(rows, out_h.at[c, pl.ds(r, CH)])


def _make_sc_aggregate(n_pad, d, nch):
    rpt = n_pad // NS
    npiece = rpt // CH
    nblk = nch // BLK

    def body(table_h, eidx_h, zf_h, agg_h, acc_sh,
             ib0, ib1, rows0, rows1, si0, si1, sg0, sg1, ss0, ss1):
        c = lax.axis_index("c")
        s = lax.axis_index("s")
        w = c * NS + s
        row0 = s * rpt
        base = w * nch

        def idx_copy(buf, sem, b):
            return pltpu.make_async_copy(
                eidx_h.at[pl.ds(base + b * BLK, BLK)], buf, sem)

        def wait_scatters(ib):
            pltpu.make_async_copy(rows0, acc_sh.at[ib.at[0, 1]], ss0).wait()
            pltpu.make_async_copy(rows1, acc_sh.at[ib.at[1, 1]], ss1).wait()

        def do_pair(ib, kk, first):
            k0 = kk * 2
            if not first:
                pltpu.make_async_copy(rows0, acc_sh.at[ib.at[k0, 1]],
                                      ss0).wait()
            g0 = pltpu.async_copy(table_h.at[ib.at[k0, 0]], rows0, sg0)
            if not first:
                pltpu.make_async_copy(rows1, acc_sh.at[ib.at[k0 + 1, 1]],
                                      ss1).wait()
            g1 = pltpu.async_copy(table_h.at[ib.at[k0 + 1, 0]], rows1, sg1)
            g0.wait()
            pltpu.async_copy(rows0, acc_sh.at[ib.at[k0, 1]], ss0, add=True)
            g1.wait()
            pltpu.async_copy(rows1, acc_sh.at[ib.at[k0 + 1, 1]], ss1,
                             add=True)

        idx_copy(ib0, si0, 0).start()
        idx_copy(ib1, si1, 1).start()
        _zero_init(zf_h, rows0, acc_sh, row0, npiece)
        plsc.subcore_barrier()

        for b in range(nblk):
            ib, si = (ib0, si0) if b % 2 == 0 else (ib1, si1)
            ibn, sin = (ib1, si1) if b % 2 == 0 else (ib0, si0)
            idx_copy(ib, si, b).wait()
            do_pair(ib, 0, first=(b == 0))
            if 0 < b < nblk - 1:
                idx_copy(ibn, sin, b + 1).start()

            @pl.loop(1, BLK // 2)
            def _(kk):
                do_pair(ib, kk, first=False)

        ib_last = ib0 if (nblk - 1) % 2 == 0 else ib1
        wait_scatters(ib_last)
        plsc.subcore_barrier()

        _copy_out(acc_sh, rows0, agg_h, c, row0, npiece)

    return pl.kernel(
        body,
        out_type=jax.ShapeDtypeStruct((NC, n_pad, d), jnp.float32),
        mesh=_MESH,
        scratch_types=[
            pltpu.VMEM_SHARED((n_pad, d), jnp.float32),
            pltpu.VMEM((BLK, 2, CH), jnp.int32),
            pltpu.VMEM((BLK, 2, CH), jnp.int32),
            pltpu.VMEM((CH, d), jnp.float32),
            pltpu.VMEM((CH, d), jnp.float32),
            pltpu.SemaphoreType.DMA,
            pltpu.SemaphoreType.DMA,
            pltpu.SemaphoreType.DMA,
            pltpu.SemaphoreType.DMA,
            pltpu.SemaphoreType.DMA,
            pltpu.SemaphoreType.DMA,
        ],
    )


def _make_sc_count(n_pad, d, nch):
    rpt = n_pad // NS
    npiece = rpt // CH
    nblk = nch // BLK

    def body(eidx_h, zf_h, on_h, cnt_h, cnt_sh,
             ib0, ib1, rows, ones_v, si0, si1, ss0, ss1):
        c = lax.axis_index("c")
        s = lax.axis_index("s")
        w = c * NS + s
        row0 = s * rpt
        base = w * nch

        def idx_copy(buf, sem, b):
            return pltpu.make_async_copy(
                eidx_h.at[pl.ds(base + b * BLK, BLK)], buf, sem)

        def do_pair(ib, kk, first):
            k0 = kk * 2
            if not first:
                pltpu.make_async_copy(ones_v, cnt_sh.at[ib.at[k0, 1]],
                                      ss0).wait()
            pltpu.async_copy(ones_v, cnt_sh.at[ib.at[k0, 1]], ss0, add=True)
            if not first:
                pltpu.make_async_copy(ones_v, cnt_sh.at[ib.at[k0 + 1, 1]],
                                      ss1).wait()
            pltpu.async_copy(ones_v, cnt_sh.at[ib.at[k0 + 1, 1]], ss1,
                             add=True)

        idx_copy(ib0, si0, 0).start()
        idx_copy(ib1, si1, 1).start()
        _zero_init(zf_h, rows, cnt_sh, row0, npiece)
        pltpu.sync_copy(on_h, ones_v)
        plsc.subcore_barrier()

        for b in range(nblk):
            ib, si = (ib0, si0) if b % 2 == 0 else (ib1, si1)
            ibn, sin = (ib1, si1) if b % 2 == 0 else (ib0, si0)
            idx_copy(ib, si, b).wait()
            do_pair(ib, 0, first=(b == 0))
            if 0 < b < nblk - 1:
                idx_copy(ibn, sin, b + 1).start()

            @pl.loop(1, BLK // 2)
            def _(kk):
                do_pair(ib, kk, first=False)

        ib_last = ib0 if (nblk - 1) % 2 == 0 else ib1
        pltpu.make_async_copy(ones_v, cnt_sh.at[ib_last.at[0, 1]], ss0).wait()
        pltpu.make_async_copy(ones_v, cnt_sh.at[ib_last.at[1, 1]], ss1).wait()
        plsc.subcore_barrier()

        _copy_out(cnt_sh, rows, cnt_h, c, row0, npiece)

    return pl.kernel(
        body,
        out_type=jax.ShapeDtypeStruct((NC, n_pad, d), jnp.float32),
        mesh=_MESH,
        scratch_types=[
            pltpu.VMEM_SHARED((n_pad, d), jnp.float32),
            pltpu.VMEM((BLK, 2, CH), jnp.int32),
            pltpu.VMEM((BLK, 2, CH), jnp.int32),
            pltpu.VMEM((CH, d), jnp.float32),
            pltpu.VMEM((CH, d), jnp.float32),
            pltpu.SemaphoreType.DMA,
            pltpu.SemaphoreType.DMA,
            pltpu.SemaphoreType.DMA,
            pltpu.SemaphoreType.DMA,
        ],
    )


def _combine_body(agg_ref, cnt_ref, h_ref, wl_ref, wr_ref, b_ref, out_ref, *,
                  relu):
    agg = agg_ref[0] + agg_ref[1]
    cnt = cnt_ref[0, :, 0:1] + cnt_ref[1, :, 0:1]
    mean = agg / jnp.maximum(cnt, 1.0)
    acc = jnp.dot(mean, wl_ref[...], preferred_element_type=jnp.float32,
                  precision=lax.Precision.HIGHEST)
    acc += jnp.dot(h_ref[...], wr_ref[...], preferred_element_type=jnp.float32,
                   precision=lax.Precision.HIGHEST)
    acc += b_ref[...]
    out_ref[...] = jnp.maximum(acc, 0.0) if relu else acc


def _combine(agg, cnt, h, wl, wr, b, relu, block):
    n, d = h.shape
    grid = (n // block,)
    return pl.pallas_call(
        functools.partial(_combine_body, relu=relu),
        grid=grid,
        in_specs=[
            pl.BlockSpec((NC, block, d), lambda i: (0, i, 0)),
            pl.BlockSpec((NC, block, d), lambda i: (0, i, 0)),
            pl.BlockSpec((block, d), lambda i: (i, 0)),
            pl.BlockSpec((d, d), lambda i: (0, 0)),
            pl.BlockSpec((d, d), lambda i: (0, 0)),
            pl.BlockSpec((1, d), lambda i: (0, 0)),
        ],
        out_specs=pl.BlockSpec((block, d), lambda i: (i, 0)),
        out_shape=jax.ShapeDtypeStruct((n, d), jnp.float32),
    )(agg, cnt, h, wl, wr, b.reshape(1, d))


@jax.jit
def kernel(x, edge_index, Wl0, Wr0, b0, Wl1, Wr1, b1):
    n, d = x.shape
    e = edge_index.shape[1]
    blk = NS * CH
    n_pad = ((n + blk - 1) // blk) * blk

    gran = NW * BLK * CH
    e_pad = ((e + gran - 1) // gran) * gran
    nch = e_pad // (NW * CH)
    pad = e_pad - e
    pad_ar = lax.iota(jnp.int32, pad)
    src = jnp.concatenate([edge_index[0], pad_ar % n])
    dst = jnp.concatenate([edge_index[1], n + pad_ar % (n_pad - n)])
    eidx = jnp.stack(
        [src.reshape(NW * nch, CH), dst.reshape(NW * nch, CH)], axis=1)

    zeros_feat = jnp.zeros((CH, d), jnp.float32)
    ones_rows = jnp.ones((CH, d), jnp.float32)

    cnt = _make_sc_count(n_pad, d, nch)(eidx, zeros_feat, ones_rows)
    agg_x = _make_sc_aggregate(n_pad, d, nch)(x, eidx, zeros_feat)
    h1 = _combine(agg_x, cnt, x, Wl0, Wr0, b0, relu=True, block=400)
    agg_h = _make_sc_aggregate(n_pad, d, nch)(h1, eidx, zeros_feat)
    out = _combine(agg_h, cnt, h1, Wl1, Wr1, b1, relu=False, block=400)
    return out


# speedup vs baseline: 2.9136x; 1.1577x over previous
import dataclasses
import functools

import jax
import jax.numpy as jnp
from jax import lax
from jax.experimental import pallas as pl
from jax.experimental.pallas import tpu as pltpu
from jax.experimental.pallas import tpu_sc as plsc

NC = 2
NS = 16
NW = NC * NS

CH = 128
BLK = 10

_MESH = plsc.VectorSubcoreMesh(core_axis_name="c", subcore_axis_name="s")


def _zero_init(zf_h, rows, acc_sh, row0, npiece):
    pltpu.sync_copy(zf_h, rows)

    @pl.loop(0, npiece)
    def _(j):
        pltpu.sync_copy(rows, acc_sh.at[pl.ds(row0 + j * CH, CH)])


def _copy_out(acc_sh, rows, out_h, c, row0, npiece):
    @pl.loop(0, npiece)
    def _(j):
        r = row0 + j * CH
        pltpu.sync_copy(acc_sh.at[pl.ds(r, CH)], rows)
        pltpu.sync_copy(rows, out_h.at[c, pl.ds(r, CH)])


def _make_sc_aggregate(n_pad, d, nch):
    rpt = n_pad // NS
    npiece = rpt // CH
    nblk = nch // BLK

    def body(table_h, eidx_h, zf_h, agg_h, acc_sh,
             ib0, ib1, rows0, rows1, si0, si1, sg0, sg1, ss0, ss1):
        c = lax.axis_index("c")
        s = lax.axis_index("s")
        w = c * NS + s
        row0 = s * rpt
        base = w * nch

        def idx_copy(buf, sem, b):
            return pltpu.make_async_copy(
                eidx_h.at[pl.ds(base + b * BLK, BLK)], buf, sem)

        def wait_scatters(ib):
            pltpu.make_async_copy(rows0, acc_sh.at[ib.at[0, 1]], ss0).wait()
            pltpu.make_async_copy(rows1, acc_sh.at[ib.at[1, 1]], ss1).wait()

        def do_pair(ib, kk, first):
            k0 = kk * 2
            if not first:
                pltpu.make_async_copy(rows0, acc_sh.at[ib.at[k0, 1]],
                                      ss0).wait()
            g0 = pltpu.async_copy(table_h.at[ib.at[k0, 0]], rows0, sg0)
            if not first:
                pltpu.make_async_copy(rows1, acc_sh.at[ib.at[k0 + 1, 1]],
                                      ss1).wait()
            g1 = pltpu.async_copy(table_h.at[ib.at[k0 + 1, 0]], rows1, sg1)
            g0.wait()
            pltpu.async_copy(rows0, acc_sh.at[ib.at[k0, 1]], ss0, add=True)
            g1.wait()
            pltpu.async_copy(rows1, acc_sh.at[ib.at[k0 + 1, 1]], ss1,
                             add=True)

        idx_copy(ib0, si0, 0).start()
        idx_copy(ib1, si1, 1).start()
        _zero_init(zf_h, rows0, acc_sh, row0, npiece)
        plsc.subcore_barrier()

        for b in range(nblk):
            ib, si = (ib0, si0) if b % 2 == 0 else (ib1, si1)
            ibn, sin = (ib1, si1) if b % 2 == 0 else (ib0, si0)
            idx_copy(ib, si, b).wait()
            do_pair(ib, 0, first=(b == 0))
            if 0 < b < nblk - 1:
                idx_copy(ibn, sin, b + 1).start()

            @pl.loop(1, BLK // 2)
            def _(kk):
                do_pair(ib, kk, first=False)

        ib_last = ib0 if (nblk - 1) % 2 == 0 else ib1
        wait_scatters(ib_last)
        plsc.subcore_barrier()

        _copy_out(acc_sh, rows0, agg_h, c, row0, npiece)

    return pl.kernel(
        body,
        out_type=jax.ShapeDtypeStruct((NC, n_pad, d), jnp.float32),
        mesh=_MESH,
        scratch_types=[
            pltpu.VMEM_SHARED((n_pad, d), jnp.float32),
            pltpu.VMEM((BLK, 2, CH), jnp.int32),
            pltpu.VMEM((BLK, 2, CH), jnp.int32),
            pltpu.VMEM((CH, d), jnp.float32),
            pltpu.VMEM((CH, d), jnp.float32),
            pltpu.SemaphoreType.DMA,
            pltpu.SemaphoreType.DMA,
            pltpu.SemaphoreType.DMA,
            pltpu.SemaphoreType.DMA,
            pltpu.SemaphoreType.DMA,
            pltpu.SemaphoreType.DMA,
        ],
    )


def _make_sc_count(n_pad, nch):
    rpt = n_pad // NS

    def body(eidx_h, z1_h, cnt_h, red_sh, idx_all, cnt_v, stage, sem, si):
        c = lax.axis_index("c")
        s = lax.axis_index("s")
        w = c * NS + s

        ld = pltpu.async_copy(eidx_h.at[pl.ds(w * nch, nch)], idx_all, si)
        pltpu.sync_copy(z1_h, cnt_v)
        ld.wait()

        ones16 = jnp.ones((16,), jnp.float32)

        @pl.loop(0, nch)
        def _(k):
            @pl.loop(0, CH // 16)
            def _(q):
                idx16 = idx_all[k, 1, pl.ds(q * 16, 16)]
                plsc.addupdate_scatter(cnt_v, [idx16], ones16)

        @pl.loop(0, NS)
        def _(j):
            pltpu.async_copy(cnt_v.at[pl.ds(j * rpt, rpt)],
                             red_sh.at[j, s], sem)

        @pl.loop(0, NS)
        def _(j):
            pltpu.make_async_copy(cnt_v.at[pl.ds(0, rpt)],
                                  red_sh.at[0, s], sem).wait()

        plsc.subcore_barrier()
        pltpu.sync_copy(red_sh.at[s], stage)

        @pl.loop(0, rpt // 16)
        def _(q):
            acc = stage[0, pl.ds(q * 16, 16)]
            for r in range(1, NS):
                acc = acc + stage[r, pl.ds(q * 16, 16)]
            cnt_v[pl.ds(q * 16, 16)] = acc

        pltpu.sync_copy(cnt_v.at[pl.ds(0, rpt)],
                        cnt_h.at[c, pl.ds(s * rpt, rpt)])

    cp = pltpu.CompilerParams()
    if "needs_layout_passes" in pltpu.CompilerParams.__dataclass_fields__:
        cp = dataclasses.replace(cp, needs_layout_passes=False)
    return pl.kernel(
        body,
        out_type=jax.ShapeDtypeStruct((NC, n_pad), jnp.float32),
        mesh=_MESH,
        compiler_params=cp,
        scratch_types=[
            pltpu.VMEM_SHARED((NS, NS, n_pad // NS), jnp.float32),
            pltpu.VMEM((nch, 2, CH), jnp.int32),
            pltpu.VMEM((n_pad,), jnp.float32),
            pltpu.VMEM((NS, n_pad // NS), jnp.float32),
            pltpu.SemaphoreType.DMA,
            pltpu.SemaphoreType.DMA,
        ],
    )


def _combine_body(agg_ref, cnt_ref, h_ref, wl_ref, wr_ref, b_ref, out_ref, *,
                  relu):
    agg = agg_ref[0] + agg_ref[1]
    cnt = cnt_ref[:, 0:1] + cnt_ref[:, 1:2]
    mean = agg / jnp.maximum(cnt, 1.0)
    acc = jnp.dot(mean, wl_ref[...], preferred_element_type=jnp.float32,
                  precision=lax.Precision.HIGHEST)
    acc += jnp.dot(h_ref[...], wr_ref[...], preferred_element_type=jnp.float32,
                   precision=lax.Precision.HIGHEST)
    acc += b_ref[...]
    out_ref[...] = jnp.maximum(acc, 0.0) if relu else acc


def _combine(agg, cnt, h, wl, wr, b, relu, block):
    n, d = h.shape
    grid = (n // block,)
    return pl.pallas_call(
        functools.partial(_combine_body, relu=relu),
        grid=grid,
        in_specs=[
            pl.BlockSpec((NC, block, d), lambda i: (0, i, 0)),
            pl.BlockSpec((block, NC), lambda i: (i, 0)),
            pl.BlockSpec((block, d), lambda i: (i, 0)),
            pl.BlockSpec((d, d), lambda i: (0, 0)),
            pl.BlockSpec((d, d), lambda i: (0, 0)),
            pl.BlockSpec((1, d), lambda i: (0, 0)),
        ],
        out_specs=pl.BlockSpec((block, d), lambda i: (i, 0)),
        out_shape=jax.ShapeDtypeStruct((n, d), jnp.float32),
    )(agg, cnt, h, wl, wr, b.reshape(1, d))


@jax.jit
def kernel(x, edge_index, Wl0, Wr0, b0, Wl1, Wr1, b1):
    n, d = x.shape
    e = edge_index.shape[1]
    blk = NS * CH
    n_pad = ((n + blk - 1) // blk) * blk

    gran = NW * BLK * CH
    e_pad = ((e + gran - 1) // gran) * gran
    nch = e_pad // (NW * CH)
    pad = e_pad - e
    pad_ar = lax.iota(jnp.int32, pad)
    src = jnp.concatenate([edge_index[0], pad_ar % n])
    dst = jnp.concatenate([edge_index[1], n + pad_ar % (n_pad - n)])
    eidx = jnp.stack(
        [src.reshape(NW * nch, CH), dst.reshape(NW * nch, CH)], axis=1)

    zeros_feat = jnp.zeros((CH, d), jnp.float32)
    zeros_1d = jnp.zeros((n_pad,), jnp.float32)

    cnt = _make_sc_count(n_pad, nch)(eidx, zeros_1d)
    cnt = cnt.T
    agg_x = _make_sc_aggregate(n_pad, d, nch)(x, eidx, zeros_feat)
    h1 = _combine(agg_x, cnt, x, Wl0, Wr0, b0, relu=True, block=400)
    agg_h = _make_sc_aggregate(n_pad, d, nch)(h1, eidx, zeros_feat)
    out = _combine(agg_h, cnt, h1, Wl1, Wr1, b1, relu=False, block=400)
    return out


# speedup vs baseline: 2.9221x; 1.0029x over previous
import dataclasses
import functools

import jax
import jax.numpy as jnp
from jax import lax
from jax.experimental import pallas as pl
from jax.experimental.pallas import tpu as pltpu
from jax.experimental.pallas import tpu_sc as plsc

NC = 2
NS = 16
NW = NC * NS

CH = 128
BLK = 10

_MESH = plsc.VectorSubcoreMesh(core_axis_name="c", subcore_axis_name="s")


def _zero_init(zf_h, rows, acc_sh, row0, npiece):
    pltpu.sync_copy(zf_h, rows)

    @pl.loop(0, npiece)
    def _(j):
        pltpu.sync_copy(rows, acc_sh.at[pl.ds(row0 + j * CH, CH)])


def _copy_out(acc_sh, rows, out_h, c, row0, npiece):
    @pl.loop(0, npiece)
    def _(j):
        r = row0 + j * CH
        pltpu.sync_copy(acc_sh.at[pl.ds(r, CH)], rows)
        pltpu.sync_copy(rows, out_h.at[c, pl.ds(r, CH)])


def _make_sc_aggregate(n_pad, d, nch):
    rpt = n_pad // NS
    npiece = rpt // CH
    nblk = nch // BLK

    def body(table_h, eidx_h, zf_h, agg_h, acc_sh,
             ib0, ib1, rows0, rows1, si0, si1, sg0, sg1, ss0, ss1):
        c = lax.axis_index("c")
        s = lax.axis_index("s")
        w = c * NS + s
        row0 = s * rpt
        base = w * nch

        def idx_copy(buf, sem, b):
            return pltpu.make_async_copy(
                eidx_h.at[pl.ds(base + b * BLK, BLK)], buf, sem)

        def wait_scatters(ib):
            pltpu.make_async_copy(rows0, acc_sh.at[ib.at[0, 1]], ss0).wait()
            pltpu.make_async_copy(rows1, acc_sh.at[ib.at[1, 1]], ss1).wait()

        def do_pair(ib, kk, first):
            k0 = kk * 2
            if not first:
                pltpu.make_async_copy(rows0, acc_sh.at[ib.at[k0, 1]],
                                      ss0).wait()
            g0 = pltpu.async_copy(table_h.at[ib.at[k0, 0]], rows0, sg0)
            if not first:
                pltpu.make_async_copy(rows1, acc_sh.at[ib.at[k0 + 1, 1]],
                                      ss1).wait()
            g1 = pltpu.async_copy(table_h.at[ib.at[k0 + 1, 0]], rows1, sg1)
            g0.wait()
            pltpu.async_copy(rows0, acc_sh.at[ib.at[k0, 1]], ss0, add=True)
            g1.wait()
            pltpu.async_copy(rows1, acc_sh.at[ib.at[k0 + 1, 1]], ss1,
                             add=True)

        idx_copy(ib0, si0, 0).start()
        idx_copy(ib1, si1, 1).start()
        _zero_init(zf_h, rows0, acc_sh, row0, npiece)
        plsc.subcore_barrier()

        for b in range(nblk):
            ib, si = (ib0, si0) if b % 2 == 0 else (ib1, si1)
            ibn, sin = (ib1, si1) if b % 2 == 0 else (ib0, si0)
            idx_copy(ib, si, b).wait()
            do_pair(ib, 0, first=(b == 0))
            if 0 < b < nblk - 1:
                idx_copy(ibn, sin, b + 1).start()

            @pl.loop(1, BLK // 2)
            def _(kk):
                do_pair(ib, kk, first=False)

        ib_last = ib0 if (nblk - 1) % 2 == 0 else ib1
        wait_scatters(ib_last)
        plsc.subcore_barrier()

        _copy_out(acc_sh, rows0, agg_h, c, row0, npiece)

    return pl.kernel(
        body,
        out_type=jax.ShapeDtypeStruct((NC, n_pad, d), jnp.float32),
        mesh=_MESH,
        scratch_types=[
            pltpu.VMEM_SHARED((n_pad, d), jnp.float32),
            pltpu.VMEM((BLK, 2, CH), jnp.int32),
            pltpu.VMEM((BLK, 2, CH), jnp.int32),
            pltpu.VMEM((CH, d), jnp.float32),
            pltpu.VMEM((CH, d), jnp.float32),
            pltpu.SemaphoreType.DMA,
            pltpu.SemaphoreType.DMA,
            pltpu.SemaphoreType.DMA,
            pltpu.SemaphoreType.DMA,
            pltpu.SemaphoreType.DMA,
            pltpu.SemaphoreType.DMA,
        ],
    )


def _make_sc_count(n_pad, nch):
    rpt = n_pad // NS

    def body(eidx_h, z1_h, cnt_h, red_sh, idx_all, cnt_v, stage, sem, si):
        c = lax.axis_index("c")
        s = lax.axis_index("s")
        w = c * NS + s

        ld = pltpu.async_copy(eidx_h.at[pl.ds(w * nch, nch)], idx_all, si)
        pltpu.sync_copy(z1_h, cnt_v)
        ld.wait()

        ones16 = jnp.ones((16,), jnp.float32)

        @pl.loop(0, nch)
        def _(k):
            @pl.loop(0, CH // 16)
            def _(q):
                idx16 = idx_all[k, 1, pl.ds(q * 16, 16)]
                plsc.addupdate_scatter(cnt_v, [idx16], ones16)

        @pl.loop(0, NS)
        def _(j):
            pltpu.async_copy(cnt_v.at[pl.ds(j * rpt, rpt)],
                             red_sh.at[j, s], sem)

        @pl.loop(0, NS)
        def _(j):
            pltpu.make_async_copy(cnt_v.at[pl.ds(0, rpt)],
                                  red_sh.at[0, s], sem).wait()

        plsc.subcore_barrier()
        pltpu.sync_copy(red_sh.at[s], stage)

        @pl.loop(0, rpt // 16)
        def _(q):
            acc = stage[0, pl.ds(q * 16, 16)]
            for r in range(1, NS):
                acc = acc + stage[r, pl.ds(q * 16, 16)]
            cnt_v[pl.ds(q * 16, 16)] = acc

        pltpu.sync_copy(cnt_v.at[pl.ds(0, rpt)],
                        cnt_h.at[c, pl.ds(s * rpt, rpt)])

    cp = pltpu.CompilerParams()
    if "needs_layout_passes" in pltpu.CompilerParams.__dataclass_fields__:
        cp = dataclasses.replace(cp, needs_layout_passes=False)
    return pl.kernel(
        body,
        out_type=jax.ShapeDtypeStruct((NC, n_pad), jnp.float32),
        mesh=_MESH,
        compiler_params=cp,
        scratch_types=[
            pltpu.VMEM_SHARED((NS, NS, n_pad // NS), jnp.float32),
            pltpu.VMEM((nch, 2, CH), jnp.int32),
            pltpu.VMEM((n_pad,), jnp.float32),
            pltpu.VMEM((NS, n_pad // NS), jnp.float32),
            pltpu.SemaphoreType.DMA,
            pltpu.SemaphoreType.DMA,
        ],
    )


def _dense_body(h_ref, wr_ref, b_ref, q_ref):
    q_ref[...] = jnp.dot(
        h_ref[...], wr_ref[...], preferred_element_type=jnp.float32,
        precision=lax.Precision.HIGHEST) + b_ref[...]


def _dense(h, wr, b, block):
    n, d = h.shape
    return pl.pallas_call(
        _dense_body,
        grid=(n // block,),
        in_specs=[
            pl.BlockSpec((block, d), lambda i: (i, 0)),
            pl.BlockSpec((d, d), lambda i: (0, 0)),
            pl.BlockSpec((1, d), lambda i: (0, 0)),
        ],
        out_specs=pl.BlockSpec((block, d), lambda i: (i, 0)),
        out_shape=jax.ShapeDtypeStruct((n, d), jnp.float32),
    )(h, wr, b.reshape(1, d))


def _combine_body(agg_ref, cnt_ref, q_ref, wl_ref, out_ref, *, relu):
    agg = agg_ref[0] + agg_ref[1]
    cnt = cnt_ref[:, 0:1] + cnt_ref[:, 1:2]
    mean = agg / jnp.maximum(cnt, 1.0)
    acc = jnp.dot(mean, wl_ref[...], preferred_element_type=jnp.float32,
                  precision=lax.Precision.HIGHEST) + q_ref[...]
    out_ref[...] = jnp.maximum(acc, 0.0) if relu else acc


def _combine(agg, cnt, q, wl, relu, block):
    n_pad = agg.shape[1]
    n, d = q.shape
    grid = (n // block,)
    return pl.pallas_call(
        functools.partial(_combine_body, relu=relu),
        grid=grid,
        in_specs=[
            pl.BlockSpec((NC, block, d), lambda i: (0, i, 0)),
            pl.BlockSpec((block, NC), lambda i: (i, 0)),
            pl.BlockSpec((block, d), lambda i: (i, 0)),
            pl.BlockSpec((d, d), lambda i: (0, 0)),
        ],
        out_specs=pl.BlockSpec((block, d), lambda i: (i, 0)),
        out_shape=jax.ShapeDtypeStruct((n, d), jnp.float32),
    )(agg, cnt, q, wl)


@jax.jit
def kernel(x, edge_index, Wl0, Wr0, b0, Wl1, Wr1, b1):
    n, d = x.shape
    e = edge_index.shape[1]
    blk = NS * CH
    n_pad = ((n + blk - 1) // blk) * blk

    gran = NW * BLK * CH
    e_pad = ((e + gran - 1) // gran) * gran
    nch = e_pad // (NW * CH)
    pad = e_pad - e
    pad_ar = lax.iota(jnp.int32, pad)
    src = jnp.concatenate([edge_index[0], pad_ar % n])
    dst = jnp.concatenate([edge_index[1], n + pad_ar % (n_pad - n)])
    eidx = jnp.stack(
        [src.reshape(NW * nch, CH), dst.reshape(NW * nch, CH)], axis=1)

    zeros_feat = jnp.zeros((CH, d), jnp.float32)
    zeros_1d = jnp.zeros((n_pad,), jnp.float32)

    q0 = _dense(x, Wr0, b0, block=400)
    cnt = _make_sc_count(n_pad, nch)(eidx, zeros_1d)
    cnt = cnt.T
    agg_x = _make_sc_aggregate(n_pad, d, nch)(x, eidx, zeros_feat)
    h1 = _combine(agg_x, cnt, q0, Wl0, relu=True, block=400)
    q1 = _dense(h1, Wr1, b1, block=400)
    agg_h = _make_sc_aggregate(n_pad, d, nch)(h1, eidx, zeros_feat)
    out = _combine(agg_h, cnt, q1, Wl1, relu=False, block=400)
    return out


# speedup vs baseline: 2.9283x; 1.0021x over previous
import dataclasses
import functools

import jax
import jax.numpy as jnp
from jax import lax
from jax.experimental import pallas as pl
from jax.experimental.pallas import tpu as pltpu
from jax.experimental.pallas import tpu_sc as plsc

NC = 2
NS = 16
NW = NC * NS

CH = 128
BLK = 20

_MESH = plsc.VectorSubcoreMesh(core_axis_name="c", subcore_axis_name="s")


def _zero_init(zf_h, rows, acc_sh, row0, npiece):
    pltpu.sync_copy(zf_h, rows)

    @pl.loop(0, npiece)
    def _(j):
        pltpu.sync_copy(rows, acc_sh.at[pl.ds(row0 + j * CH, CH)])


def _copy_out(acc_sh, rows, out_h, c, row0, npiece):
    @pl.loop(0, npiece)
    def _(j):
        r = row0 + j * CH
        pltpu.sync_copy(acc_sh.at[pl.ds(r, CH)], rows)
        pltpu.sync_copy(rows, out_h.at[c, pl.ds(r, CH)])


def _make_sc_aggregate(n_pad, d, nch):
    rpt = n_pad // NS
    npiece = rpt // CH
    nblk = nch // BLK

    def body(table_h, eidx_h, zf_h, agg_h, acc_sh,
             ib0, ib1, rows0, rows1, si0, si1, sg0, sg1, ss0, ss1):
        c = lax.axis_index("c")
        s = lax.axis_index("s")
        w = c * NS + s
        row0 = s * rpt
        base = w * nch

        def idx_copy(buf, sem, b):
            return pltpu.make_async_copy(
                eidx_h.at[pl.ds(base + b * BLK, BLK)], buf, sem)

        def wait_scatters(ib):
            pltpu.make_async_copy(rows0, acc_sh.at[ib.at[0, 1]], ss0).wait()
            pltpu.make_async_copy(rows1, acc_sh.at[ib.at[1, 1]], ss1).wait()

        def do_pair(ib, kk, first):
            k0 = kk * 2
            if not first:
                pltpu.make_async_copy(rows0, acc_sh.at[ib.at[k0, 1]],
                                      ss0).wait()
            g0 = pltpu.async_copy(table_h.at[ib.at[k0, 0]], rows0, sg0)
            if not first:
                pltpu.make_async_copy(rows1, acc_sh.at[ib.at[k0 + 1, 1]],
                                      ss1).wait()
            g1 = pltpu.async_copy(table_h.at[ib.at[k0 + 1, 0]], rows1, sg1)
            g0.wait()
            pltpu.async_copy(rows0, acc_sh.at[ib.at[k0, 1]], ss0, add=True)
            g1.wait()
            pltpu.async_copy(rows1, acc_sh.at[ib.at[k0 + 1, 1]], ss1,
                             add=True)

        idx_copy(ib0, si0, 0).start()
        idx_copy(ib1, si1, 1).start()
        _zero_init(zf_h, rows0, acc_sh, row0, npiece)
        plsc.subcore_barrier()

        for b in range(nblk):
            ib, si = (ib0, si0) if b % 2 == 0 else (ib1, si1)
            ibn, sin = (ib1, si1) if b % 2 == 0 else (ib0, si0)
            idx_copy(ib, si, b).wait()
            do_pair(ib, 0, first=(b == 0))
            if 0 < b < nblk - 1:
                idx_copy(ibn, sin, b + 1).start()

            @pl.loop(1, BLK // 2)
            def _(kk):
                do_pair(ib, kk, first=False)

        ib_last = ib0 if (nblk - 1) % 2 == 0 else ib1
        wait_scatters(ib_last)
        plsc.subcore_barrier()

        _copy_out(acc_sh, rows0, agg_h, c, row0, npiece)

    return pl.kernel(
        body,
        out_type=jax.ShapeDtypeStruct((NC, n_pad, d), jnp.float32),
        mesh=_MESH,
        scratch_types=[
            pltpu.VMEM_SHARED((n_pad, d), jnp.float32),
            pltpu.VMEM((BLK, 2, CH), jnp.int32),
            pltpu.VMEM((BLK, 2, CH), jnp.int32),
            pltpu.VMEM((CH, d), jnp.float32),
            pltpu.VMEM((CH, d), jnp.float32),
            pltpu.SemaphoreType.DMA,
            pltpu.SemaphoreType.DMA,
            pltpu.SemaphoreType.DMA,
            pltpu.SemaphoreType.DMA,
            pltpu.SemaphoreType.DMA,
            pltpu.SemaphoreType.DMA,
        ],
    )


def _make_sc_count(n_pad, nch):
    rpt = n_pad // NS

    def body(eidx_h, z1_h, cnt_h, red_sh, idx_all, cnt_v, stage, sem, si):
        c = lax.axis_index("c")
        s = lax.axis_index("s")
        w = c * NS + s

        ld = pltpu.async_copy(eidx_h.at[pl.ds(w * nch, nch)], idx_all, si)
        pltpu.sync_copy(z1_h, cnt_v)
        ld.wait()

        ones16 = jnp.ones((16,), jnp.float32)

        @pl.loop(0, nch)
        def _(k):
            @pl.loop(0, CH // 16)
            def _(q):
                idx16 = idx_all[k, 1, pl.ds(q * 16, 16)]
                plsc.addupdate_scatter(cnt_v, [idx16], ones16)

        @pl.loop(0, NS)
        def _(j):
            pltpu.async_copy(cnt_v.at[pl.ds(j * rpt, rpt)],
                             red_sh.at[j, s], sem)

        @pl.loop(0, NS)
        def _(j):
            pltpu.make_async_copy(cnt_v.at[pl.ds(0, rpt)],
                                  red_sh.at[0, s], sem).wait()

        plsc.subcore_barrier()
        pltpu.sync_copy(red_sh.at[s], stage)

        @pl.loop(0, rpt // 16)
        def _(q):
            acc = stage[0, pl.ds(q * 16, 16)]
            for r in range(1, NS):
                acc = acc + stage[r, pl.ds(q * 16, 16)]
            cnt_v[pl.ds(q * 16, 16)] = acc

        pltpu.sync_copy(cnt_v.at[pl.ds(0, rpt)],
                        cnt_h.at[c, pl.ds(s * rpt, rpt)])

    cp = pltpu.CompilerParams()
    if "needs_layout_passes" in pltpu.CompilerParams.__dataclass_fields__:
        cp = dataclasses.replace(cp, needs_layout_passes=False)
    return pl.kernel(
        body,
        out_type=jax.ShapeDtypeStruct((NC, n_pad), jnp.float32),
        mesh=_MESH,
        compiler_params=cp,
        scratch_types=[
            pltpu.VMEM_SHARED((NS, NS, n_pad // NS), jnp.float32),
            pltpu.VMEM((nch, 2, CH), jnp.int32),
            pltpu.VMEM((n_pad,), jnp.float32),
            pltpu.VMEM((NS, n_pad // NS), jnp.float32),
            pltpu.SemaphoreType.DMA,
            pltpu.SemaphoreType.DMA,
        ],
    )


def _dense_body(h_ref, wr_ref, b_ref, q_ref):
    q_ref[...] = jnp.dot(
        h_ref[...], wr_ref[...], preferred_element_type=jnp.float32,
        precision=lax.Precision.HIGHEST) + b_ref[...]


def _dense(h, wr, b, block):
    n, d = h.shape
    return pl.pallas_call(
        _dense_body,
        grid=(n // block,),
        in_specs=[
            pl.BlockSpec((block, d), lambda i: (i, 0)),
            pl.BlockSpec((d, d), lambda i: (0, 0)),
            pl.BlockSpec((1, d), lambda i: (0, 0)),
        ],
        out_specs=pl.BlockSpec((block, d), lambda i: (i, 0)),
        out_shape=jax.ShapeDtypeStruct((n, d), jnp.float32),
    )(h, wr, b.reshape(1, d))


def _combine_body(agg_ref, cnt_ref, q_ref, wl_ref, out_ref, *, relu):
    agg = agg_ref[0] + agg_ref[1]
    cnt = cnt_ref[:, 0:1] + cnt_ref[:, 1:2]
    mean = agg / jnp.maximum(cnt, 1.0)
    acc = jnp.dot(mean, wl_ref[...], preferred_element_type=jnp.float32,
                  precision=lax.Precision.HIGHEST) + q_ref[...]
    out_ref[...] = jnp.maximum(acc, 0.0) if relu else acc


def _combine(agg, cnt, q, wl, relu, block):
    n_pad = agg.shape[1]
    n, d = q.shape
    grid = (n // block,)
    return pl.pallas_call(
        functools.partial(_combine_body, relu=relu),
        grid=grid,
        in_specs=[
            pl.BlockSpec((NC, block, d), lambda i: (0, i, 0)),
            pl.BlockSpec((block, NC), lambda i: (i, 0)),
            pl.BlockSpec((block, d), lambda i: (i, 0)),
            pl.BlockSpec((d, d), lambda i: (0, 0)),
        ],
        out_specs=pl.BlockSpec((block, d), lambda i: (i, 0)),
        out_shape=jax.ShapeDtypeStruct((n, d), jnp.float32),
    )(agg, cnt, q, wl)


@jax.jit
def kernel(x, edge_index, Wl0, Wr0, b0, Wl1, Wr1, b1):
    n, d = x.shape
    e = edge_index.shape[1]
    blk = NS * CH
    n_pad = ((n + blk - 1) // blk) * blk

    gran = NW * BLK * CH
    e_pad = ((e + gran - 1) // gran) * gran
    nch = e_pad // (NW * CH)
    pad = e_pad - e
    pad_ar = lax.iota(jnp.int32, pad)
    src = jnp.concatenate([edge_index[0], pad_ar % n])
    dst = jnp.concatenate([edge_index[1], n + pad_ar % (n_pad - n)])
    eidx = jnp.stack(
        [src.reshape(NW * nch, CH), dst.reshape(NW * nch, CH)], axis=1)

    zeros_feat = jnp.zeros((CH, d), jnp.float32)
    zeros_1d = jnp.zeros((n_pad,), jnp.float32)

    q0 = _dense(x, Wr0, b0, block=400)
    cnt = _make_sc_count(n_pad, nch)(eidx, zeros_1d)
    cnt = cnt.T
    agg_x = _make_sc_aggregate(n_pad, d, nch)(x, eidx, zeros_feat)
    h1 = _combine(agg_x, cnt, q0, Wl0, relu=True, block=400)
    q1 = _dense(h1, Wr1, b1, block=400)
    agg_h = _make_sc_aggregate(n_pad, d, nch)(h1, eidx, zeros_feat)
    out = _combine(agg_h, cnt, q1, Wl1, relu=False, block=400)
    return out
